# baseline (speedup 1.0000x reference)
import functools

import jax
import jax.numpy as jnp
from jax import lax
from jax.experimental import pallas as pl
from jax.experimental.pallas import tpu as pltpu
from jax.experimental.pallas import tpu_sc as plsc

_WORLD_SIZE = 4
_RANK = 0
_MARGIN = 1.0
_CHUNK = 512


def _mine_idx_kernel(b_ref, pos_ref, neg_ref, min_ref, chunk_ref):
    n, d = b_ref.shape
    m = pos_ref.shape[0]
    a = b_ref[:m, :]
    n_chunks = n // _CHUNK
    n_pos_chunks = m // _CHUNK
    a_m2 = a * (-2.0)
    diag_mask = (lax.broadcasted_iota(jnp.int32, (_CHUNK, _CHUNK), 0) // 16
                 == lax.broadcasted_iota(jnp.int32, (_CHUNK, _CHUNK), 1) // 16)
    lane512 = lax.broadcasted_iota(jnp.int32, (_CHUNK, _CHUNK), 1)

    def upd_region(t_region, c, lo, hi):
        old = min_ref[lo:hi, :]
        upd = t_region < old
        min_ref[lo:hi, :] = jnp.where(upd, t_region, old)
        chunk_ref[lo:hi, :] = jnp.where(upd, c, chunk_ref[lo:hi, :])

    ones_row = jnp.ones((1, d), dtype=a.dtype)

    def dist_chunk(c):
        bchunk = b_ref[pl.ds(c * _CHUNK, _CHUNK), :]
        b2 = lax.dot_general(ones_row, bchunk * bchunk,
                             dimension_numbers=(((1,), (1,)), ((), ())),
                             preferred_element_type=jnp.float32)
        return lax.dot_general(a_m2, bchunk,
                               dimension_numbers=(((1,), (1,)), ((), ())),
                               preferred_element_type=jnp.float32) + b2

    min_ref[...] = jnp.full((m, _CHUNK), jnp.inf, jnp.float32)
    chunk_ref[...] = jnp.zeros((m, _CHUNK), jnp.int32)
    for c in range(n_pos_chunks):
        t = dist_chunk(c)
        lo, hi = c * _CHUNK, (c + 1) * _CHUNK
        slab = lax.slice(t, (lo, 0), (hi, _CHUNK))
        masked = jnp.where(diag_mask, slab, -jnp.inf)
        vmax = jnp.max(masked, axis=1, keepdims=True)
        pos_j = jnp.min(jnp.where(masked == vmax, lane512, n),
                        axis=1, keepdims=True)
        pos_ref[lo:hi, :] = pos_j + c * _CHUNK
        slab_neg = jnp.where(diag_mask, jnp.inf, slab)
        if c > 0:
            upd_region(lax.slice(t, (0, 0), (lo, _CHUNK)), c, 0, lo)
        upd_region(slab_neg, c, lo, hi)
        if hi < m:
            upd_region(lax.slice(t, (hi, 0), (m, _CHUNK)), c, hi, m)

    def neg_body(c, _):
        upd_region(dist_chunk(c), c, 0, m)
        return _

    lax.fori_loop(n_pos_chunks, n_chunks, neg_body, 0)

    run_min = min_ref[...]
    lane = lax.broadcasted_iota(jnp.int32, (m, _CHUNK), 1)
    vmin = jnp.min(run_min, axis=1, keepdims=True)
    j_star = jnp.min(jnp.where(run_min == vmin, lane, n),
                     axis=1, keepdims=True)
    c_star = jnp.sum(jnp.where(lane == j_star, chunk_ref[...], 0),
                     axis=1, keepdims=True)
    neg_ref[...] = c_star * _CHUNK + j_star


def _sqrt16(x):
    i = plsc.bitcast(x, jnp.int32)
    y = plsc.bitcast(jnp.int32(0x1FBD1DF5) + (i >> 1), jnp.float32)
    for _ in range(4):
        y = 0.5 * (y + x / y)
    return y


@functools.cache
def _make_sc_loss(m, d, b_per_w, nc):
    mesh = plsc.VectorSubcoreMesh(core_axis_name="c", subcore_axis_name="s")

    @functools.partial(
        pl.kernel, mesh=mesh,
        compiler_params=pltpu.CompilerParams(needs_layout_passes=False),
        out_type=jax.ShapeDtypeStruct((m,), jnp.float32),
        scratch_types=[
            pltpu.VMEM((b_per_w,), jnp.int32),
            pltpu.VMEM((b_per_w,), jnp.int32),
            pltpu.VMEM((b_per_w, d), jnp.float32),
            pltpu.VMEM((b_per_w, d), jnp.float32),
            pltpu.VMEM((b_per_w, d), jnp.float32),
            pltpu.VMEM((b_per_w,), jnp.float32),
            pltpu.SemaphoreType.DMA,
            pltpu.SemaphoreType.DMA,
        ],
    )
    def sc_loss(all_hbm, pidx_hbm, nidx_hbm, out_hbm,
                pidx_v, nidx_v, a_v, p_v, n_v, loss_v, sem1, sem2):
        wid = lax.axis_index("s") * nc + lax.axis_index("c")
        base = wid * b_per_w
        with jax.named_scope("idx_dma"):
            pltpu.sync_copy(pidx_hbm.at[pl.ds(base, b_per_w)], pidx_v)
            pltpu.sync_copy(nidx_hbm.at[pl.ds(base, b_per_w)], nidx_v)
        with jax.named_scope("gather_dma"):
            cp = pltpu.async_copy(all_hbm.at[pidx_v], p_v, sem1)
            cn = pltpu.async_copy(all_hbm.at[nidx_v], n_v, sem2)
            pltpu.sync_copy(all_hbm.at[pl.ds(base, b_per_w)], a_v)
            cp.wait()
            cn.wait()

        lanes = lax.iota(jnp.int32, 16)
        nb = b_per_w // 16
        rows_b = [b * 16 + lanes for b in range(nb)]
        zeros = jnp.zeros((16,), jnp.float32)

        def body(j, carry):
            cols = jnp.full((16,), 0, jnp.int32) + j
            out = []
            for b in range(nb):
                ap2, an2 = carry[b]
                va = plsc.load_gather(a_v, [rows_b[b], cols])
                vp = plsc.load_gather(p_v, [rows_b[b], cols])
                vn = plsc.load_gather(n_v, [rows_b[b], cols])
                dp = va - vp
                dn = va - vn
                out.append((ap2 + dp * dp, an2 + dn * dn))
            return tuple(out)

        with jax.named_scope("compute"):
            acc = lax.fori_loop(0, d, body,
                                tuple((zeros, zeros) for _ in range(nb)))
            for b in range(nb):
                ap2, an2 = acc[b]
                ap = _sqrt16(ap2 + 1e-12)
                an = _sqrt16(an2 + 1e-12)
                loss_v[pl.ds(b * 16, 16)] = jnp.maximum(ap - an + _MARGIN, 0.0)

        with jax.named_scope("out_dma"):
            pltpu.sync_copy(loss_v, out_hbm.at[pl.ds(base, b_per_w)])

    return sc_loss


@jax.jit
def kernel(batch):
    cluster_amnt, cluster_size, d = batch.shape
    base = cluster_amnt // _WORLD_SIZE
    rem = cluster_amnt % _WORLD_SIZE
    start = _RANK * base + min(_RANK, rem)
    cnt = base + (1 if _RANK < rem else 0)
    all_embeds = batch.reshape(-1, d)
    m = cnt * cluster_size
    assert start == 0, "kernel specialized for rank 0 (matches reference)"
    pos_idx, neg_idx = pl.pallas_call(
        _mine_idx_kernel,
        out_shape=(jax.ShapeDtypeStruct((m, 1), jnp.int32),
                   jax.ShapeDtypeStruct((m, 1), jnp.int32)),
        scratch_shapes=[pltpu.VMEM((m, _CHUNK), jnp.float32),
                        pltpu.VMEM((m, _CHUNK), jnp.int32)],
    )(all_embeds)

    info = plsc.get_sparse_core_info()
    nw = info.num_cores * info.num_subcores
    sc_loss = _make_sc_loss(m, d, m // nw, info.num_cores)
    return sc_loss(all_embeds, pos_idx[:, 0], neg_idx[:, 0])


# speedup vs baseline: 1.4756x; 1.4756x over previous
import functools

import jax
import jax.numpy as jnp
from jax import lax
from jax.experimental import pallas as pl
from jax.experimental.pallas import tpu as pltpu
from jax.experimental.pallas import tpu_sc as plsc

_WORLD_SIZE = 4
_RANK = 0
_MARGIN = 1.0
_CHUNK = 512


def _mine_idx_kernel(b_ref, pos_ref, neg_ref, min_ref, chunk_ref):
    n, d = b_ref.shape
    m = pos_ref.shape[0]
    a = b_ref[:m, :]
    n_chunks = n // _CHUNK
    n_pos_chunks = m // _CHUNK
    a_m2 = a * (-2.0)
    diag_mask = (lax.broadcasted_iota(jnp.int32, (_CHUNK, _CHUNK), 0) // 16
                 == lax.broadcasted_iota(jnp.int32, (_CHUNK, _CHUNK), 1) // 16)
    lane512 = lax.broadcasted_iota(jnp.int32, (_CHUNK, _CHUNK), 1)

    def upd_region(t_region, c, lo, hi):
        old = min_ref[lo:hi, :]
        upd = t_region < old
        min_ref[lo:hi, :] = jnp.where(upd, t_region, old)
        chunk_ref[lo:hi, :] = jnp.where(upd, c, chunk_ref[lo:hi, :])

    ones_row = jnp.ones((1, d), dtype=a.dtype)

    def dist_chunk(c):
        bchunk = b_ref[pl.ds(c * _CHUNK, _CHUNK), :]
        b2 = lax.dot_general(ones_row, bchunk * bchunk,
                             dimension_numbers=(((1,), (1,)), ((), ())),
                             preferred_element_type=jnp.float32)
        return lax.dot_general(a_m2, bchunk,
                               dimension_numbers=(((1,), (1,)), ((), ())),
                               preferred_element_type=jnp.float32) + b2

    min_ref[...] = jnp.full((m, _CHUNK), jnp.inf, jnp.float32)
    chunk_ref[...] = jnp.zeros((m, _CHUNK), jnp.int32)
    for c in range(n_pos_chunks):
        t = dist_chunk(c)
        lo, hi = c * _CHUNK, (c + 1) * _CHUNK
        slab = lax.slice(t, (lo, 0), (hi, _CHUNK))
        masked = jnp.where(diag_mask, slab, -jnp.inf)
        vmax = jnp.max(masked, axis=1, keepdims=True)
        pos_j = jnp.min(jnp.where(masked == vmax, lane512, n),
                        axis=1, keepdims=True)
        pos_ref[lo:hi, :] = pos_j + c * _CHUNK
        slab_neg = jnp.where(diag_mask, jnp.inf, slab)
        if c > 0:
            upd_region(lax.slice(t, (0, 0), (lo, _CHUNK)), c, 0, lo)
        upd_region(slab_neg, c, lo, hi)
        if hi < m:
            upd_region(lax.slice(t, (hi, 0), (m, _CHUNK)), c, hi, m)

    def neg_body(c, _):
        upd_region(dist_chunk(c), c, 0, m)
        return _

    lax.fori_loop(n_pos_chunks, n_chunks, neg_body, 0)

    run_min = min_ref[...]
    lane = lax.broadcasted_iota(jnp.int32, (m, _CHUNK), 1)
    vmin = jnp.min(run_min, axis=1, keepdims=True)
    j_star = jnp.min(jnp.where(run_min == vmin, lane, n),
                     axis=1, keepdims=True)
    c_star = jnp.sum(jnp.where(lane == j_star, chunk_ref[...], 0),
                     axis=1, keepdims=True)
    neg_ref[...] = c_star * _CHUNK + j_star


def _sqrt16(x):
    i = plsc.bitcast(x, jnp.int32)
    y = plsc.bitcast(jnp.int32(0x1FBD1DF5) + (i >> 1), jnp.float32)
    for _ in range(4):
        y = 0.5 * (y + x / y)
    return y


@functools.cache
def _make_sc_loss(m, d, b_per_w, nc):
    mesh = plsc.VectorSubcoreMesh(core_axis_name="c", subcore_axis_name="s")

    @functools.partial(
        pl.kernel, mesh=mesh,
        compiler_params=pltpu.CompilerParams(needs_layout_passes=False),
        out_type=jax.ShapeDtypeStruct((m,), jnp.float32),
        scratch_types=[
            pltpu.VMEM((b_per_w,), jnp.int32),
            pltpu.VMEM((b_per_w,), jnp.int32),
            pltpu.VMEM((b_per_w, d), jnp.float32),
            pltpu.VMEM((b_per_w, d), jnp.float32),
            pltpu.VMEM((b_per_w, d), jnp.float32),
            pltpu.VMEM((b_per_w,), jnp.float32),
            pltpu.SemaphoreType.DMA,
            pltpu.SemaphoreType.DMA,
        ],
    )
    def sc_loss(all_hbm, pidx_hbm, nidx_hbm, out_hbm,
                pidx_v, nidx_v, a_v, p_v, n_v, loss_v, sem1, sem2):
        wid = lax.axis_index("s") * nc + lax.axis_index("c")
        base = wid * b_per_w
        with jax.named_scope("idx_dma"):
            pltpu.sync_copy(pidx_hbm.at[pl.ds(base, b_per_w)], pidx_v)
            pltpu.sync_copy(nidx_hbm.at[pl.ds(base, b_per_w)], nidx_v)
        with jax.named_scope("gather_dma"):
            cp = pltpu.async_copy(all_hbm.at[pidx_v], p_v, sem1)
            cn = pltpu.async_copy(all_hbm.at[nidx_v], n_v, sem2)
            pltpu.sync_copy(all_hbm.at[pl.ds(base, b_per_w)], a_v)
            cp.wait()
            cn.wait()

        lanes = lax.iota(jnp.int32, 16)
        nb = b_per_w // 16
        rows_b = [b * 16 + lanes for b in range(nb)]
        zeros = jnp.zeros((16,), jnp.float32)

        def body(j, carry):
            cols = (lanes + j) & (d - 1)
            out = []
            for b in range(nb):
                ap2, an2 = carry[b]
                va = plsc.load_gather(a_v, [rows_b[b], cols])
                vp = plsc.load_gather(p_v, [rows_b[b], cols])
                vn = plsc.load_gather(n_v, [rows_b[b], cols])
                dp = va - vp
                dn = va - vn
                out.append((ap2 + dp * dp, an2 + dn * dn))
            return tuple(out)

        with jax.named_scope("compute"):
            acc = lax.fori_loop(0, d, body,
                                tuple((zeros, zeros) for _ in range(nb)))
            for b in range(nb):
                ap2, an2 = acc[b]
                ap = _sqrt16(ap2 + 1e-12)
                an = _sqrt16(an2 + 1e-12)
                loss_v[pl.ds(b * 16, 16)] = jnp.maximum(ap - an + _MARGIN, 0.0)

        with jax.named_scope("out_dma"):
            pltpu.sync_copy(loss_v, out_hbm.at[pl.ds(base, b_per_w)])

    return sc_loss


@jax.jit
def kernel(batch):
    cluster_amnt, cluster_size, d = batch.shape
    base = cluster_amnt // _WORLD_SIZE
    rem = cluster_amnt % _WORLD_SIZE
    start = _RANK * base + min(_RANK, rem)
    cnt = base + (1 if _RANK < rem else 0)
    all_embeds = batch.reshape(-1, d)
    m = cnt * cluster_size
    assert start == 0, "kernel specialized for rank 0 (matches reference)"
    pos_idx, neg_idx = pl.pallas_call(
        _mine_idx_kernel,
        out_shape=(jax.ShapeDtypeStruct((m, 1), jnp.int32),
                   jax.ShapeDtypeStruct((m, 1), jnp.int32)),
        scratch_shapes=[pltpu.VMEM((m, _CHUNK), jnp.float32),
                        pltpu.VMEM((m, _CHUNK), jnp.int32)],
    )(all_embeds)

    info = plsc.get_sparse_core_info()
    nw = info.num_cores * info.num_subcores
    sc_loss = _make_sc_loss(m, d, m // nw, info.num_cores)
    return sc_loss(all_embeds, pos_idx[:, 0], neg_idx[:, 0])


# speedup vs baseline: 1.6288x; 1.1038x over previous
import functools

import jax
import jax.numpy as jnp
from jax import lax
from jax.experimental import pallas as pl
from jax.experimental.pallas import tpu as pltpu
from jax.experimental.pallas import tpu_sc as plsc

_WORLD_SIZE = 4
_RANK = 0
_MARGIN = 1.0
_CHUNK = 512


def _mine_idx_kernel(b_ref, pos_ref, neg_ref, min_ref, chunk_ref):
    n, d = b_ref.shape
    m = pos_ref.shape[0] * pos_ref.shape[1]
    a = b_ref[:m, :]
    n_chunks = n // _CHUNK
    n_pos_chunks = m // _CHUNK
    a_m2 = a * (-2.0)
    diag_mask = (lax.broadcasted_iota(jnp.int32, (_CHUNK, _CHUNK), 0) // 16
                 == lax.broadcasted_iota(jnp.int32, (_CHUNK, _CHUNK), 1) // 16)
    lane512 = lax.broadcasted_iota(jnp.int32, (_CHUNK, _CHUNK), 1)

    def upd_region(t_region, c, lo, hi):
        old = min_ref[lo:hi, :]
        upd = t_region < old
        min_ref[lo:hi, :] = jnp.where(upd, t_region, old)
        chunk_ref[lo:hi, :] = jnp.where(upd, c, chunk_ref[lo:hi, :])

    ones_row = jnp.ones((1, d), dtype=a.dtype)

    def dist_chunk(c):
        bchunk = b_ref[pl.ds(c * _CHUNK, _CHUNK), :]
        b2 = lax.dot_general(ones_row, bchunk * bchunk,
                             dimension_numbers=(((1,), (1,)), ((), ())),
                             preferred_element_type=jnp.float32)
        return lax.dot_general(a_m2, bchunk,
                               dimension_numbers=(((1,), (1,)), ((), ())),
                               preferred_element_type=jnp.float32) + b2

    min_ref[...] = jnp.full((m, _CHUNK), jnp.inf, jnp.float32)
    chunk_ref[...] = jnp.zeros((m, _CHUNK), jnp.int32)
    for c in range(n_pos_chunks):
        t = dist_chunk(c)
        lo, hi = c * _CHUNK, (c + 1) * _CHUNK
        slab = lax.slice(t, (lo, 0), (hi, _CHUNK))
        masked = jnp.where(diag_mask, slab, -jnp.inf)
        vmax = jnp.max(masked, axis=1, keepdims=True)
        pos_j = jnp.min(jnp.where(masked == vmax, lane512, n),
                        axis=1, keepdims=True)
        pos_ref[c * (_CHUNK // 128):(c + 1) * (_CHUNK // 128), :] = (
            (pos_j + c * _CHUNK).reshape(_CHUNK // 128, 128))
        slab_neg = jnp.where(diag_mask, jnp.inf, slab)
        if c > 0:
            upd_region(lax.slice(t, (0, 0), (lo, _CHUNK)), c, 0, lo)
        upd_region(slab_neg, c, lo, hi)
        if hi < m:
            upd_region(lax.slice(t, (hi, 0), (m, _CHUNK)), c, hi, m)

    def neg_body(c, _):
        upd_region(dist_chunk(c), c, 0, m)
        return _

    lax.fori_loop(n_pos_chunks, n_chunks, neg_body, 0)

    run_min = min_ref[...]
    lane = lax.broadcasted_iota(jnp.int32, (m, _CHUNK), 1)
    vmin = jnp.min(run_min, axis=1, keepdims=True)
    j_star = jnp.min(jnp.where(run_min == vmin, lane, n),
                     axis=1, keepdims=True)
    c_star = jnp.sum(jnp.where(lane == j_star, chunk_ref[...], 0),
                     axis=1, keepdims=True)
    neg_ref[...] = (c_star * _CHUNK + j_star).reshape(m // 128, 128)


def _sqrt16(x):
    i = plsc.bitcast(x, jnp.int32)
    y = plsc.bitcast(jnp.int32(0x1FBD1DF5) + (i >> 1), jnp.float32)
    for _ in range(4):
        y = 0.5 * (y + x / y)
    return y


@functools.cache
def _make_sc_loss(m, d, b_per_w, nc):
    mesh = plsc.VectorSubcoreMesh(core_axis_name="c", subcore_axis_name="s")

    @functools.partial(
        pl.kernel, mesh=mesh,
        compiler_params=pltpu.CompilerParams(needs_layout_passes=False),
        out_type=jax.ShapeDtypeStruct((m,), jnp.float32),
        scratch_types=[
            pltpu.VMEM((b_per_w,), jnp.int32),
            pltpu.VMEM((b_per_w,), jnp.int32),
            pltpu.VMEM((b_per_w, d), jnp.float32),
            pltpu.VMEM((b_per_w, d), jnp.float32),
            pltpu.VMEM((b_per_w, d), jnp.float32),
            pltpu.VMEM((b_per_w,), jnp.float32),
            pltpu.SemaphoreType.DMA,
            pltpu.SemaphoreType.DMA,
        ],
    )
    def sc_loss(all_hbm, pidx_hbm, nidx_hbm, out_hbm,
                pidx_v, nidx_v, a_v, p_v, n_v, loss_v, sem1, sem2):
        wid = lax.axis_index("s") * nc + lax.axis_index("c")
        base = wid * b_per_w
        with jax.named_scope("idx_dma"):
            pltpu.sync_copy(pidx_hbm.at[pl.ds(base, b_per_w)], pidx_v)
            pltpu.sync_copy(nidx_hbm.at[pl.ds(base, b_per_w)], nidx_v)
        with jax.named_scope("gather_dma"):
            cp = pltpu.async_copy(all_hbm.at[pidx_v], p_v, sem1)
            cn = pltpu.async_copy(all_hbm.at[nidx_v], n_v, sem2)
            pltpu.sync_copy(all_hbm.at[pl.ds(base, b_per_w)], a_v)
            cp.wait()
            cn.wait()

        lanes = lax.iota(jnp.int32, 16)
        nb = b_per_w // 16
        rows_b = [b * 16 + lanes for b in range(nb)]
        zeros = jnp.zeros((16,), jnp.float32)

        def body(j, carry):
            cols = (lanes + j) & (d - 1)
            out = []
            for b in range(nb):
                ap2, an2 = carry[b]
                va = plsc.load_gather(a_v, [rows_b[b], cols])
                vp = plsc.load_gather(p_v, [rows_b[b], cols])
                vn = plsc.load_gather(n_v, [rows_b[b], cols])
                dp = va - vp
                dn = va - vn
                out.append((ap2 + dp * dp, an2 + dn * dn))
            return tuple(out)

        with jax.named_scope("compute"):
            acc = lax.fori_loop(0, d, body,
                                tuple((zeros, zeros) for _ in range(nb)))
            for b in range(nb):
                ap2, an2 = acc[b]
                ap = _sqrt16(ap2 + 1e-12)
                an = _sqrt16(an2 + 1e-12)
                loss_v[pl.ds(b * 16, 16)] = jnp.maximum(ap - an + _MARGIN, 0.0)

        with jax.named_scope("out_dma"):
            pltpu.sync_copy(loss_v, out_hbm.at[pl.ds(base, b_per_w)])

    return sc_loss


@jax.jit
def kernel(batch):
    cluster_amnt, cluster_size, d = batch.shape
    base = cluster_amnt // _WORLD_SIZE
    rem = cluster_amnt % _WORLD_SIZE
    start = _RANK * base + min(_RANK, rem)
    cnt = base + (1 if _RANK < rem else 0)
    all_embeds = batch.reshape(-1, d)
    m = cnt * cluster_size
    assert start == 0, "kernel specialized for rank 0 (matches reference)"
    pos_idx, neg_idx = pl.pallas_call(
        _mine_idx_kernel,
        out_shape=(jax.ShapeDtypeStruct((m // 128, 128), jnp.int32),
                   jax.ShapeDtypeStruct((m // 128, 128), jnp.int32)),
        scratch_shapes=[pltpu.VMEM((m, _CHUNK), jnp.float32),
                        pltpu.VMEM((m, _CHUNK), jnp.int32)],
    )(all_embeds)

    info = plsc.get_sparse_core_info()
    nw = info.num_cores * info.num_subcores
    sc_loss = _make_sc_loss(m, d, m // nw, info.num_cores)
    return sc_loss(all_embeds, pos_idx.reshape(m), neg_idx.reshape(m))
